# baseline (speedup 1.0000x reference)
import functools

import jax
import jax.numpy as jnp
from jax import lax
from jax.experimental import pallas as pl
from jax.experimental.pallas import tpu as pltpu
from jax.experimental.pallas import tpu_sc as plsc


def _make_gather(B, D, NC, NS):
    NW = NC * NS
    b_per_w = B // NW
    mesh = plsc.VectorSubcoreMesh(core_axis_name="c", subcore_axis_name="s")

    @functools.partial(
        pl.kernel,
        mesh=mesh,
        compiler_params=pltpu.CompilerParams(use_tc_tiling_on_sc=False),
        out_type=(
            jax.ShapeDtypeStruct((B, D), jnp.float32),
            jax.ShapeDtypeStruct((B, D), jnp.float32),
        ),
        scratch_types=[
            pltpu.VMEM((b_per_w,), jnp.int32),
            pltpu.VMEM((b_per_w,), jnp.int32),
            pltpu.VMEM((b_per_w, D), jnp.float32),
            pltpu.VMEM((b_per_w, D), jnp.float32),
            pltpu.SemaphoreType.DMA,
            pltpu.SemaphoreType.DMA,
        ],
    )
    def gather(uid_hbm, mid_hbm, ut_hbm, mt_hbm, uout_hbm, mout_hbm,
               uidx_v, midx_v, urows_v, mrows_v, usem, msem):
        wid = lax.axis_index("s") * NC + lax.axis_index("c")
        base = wid * b_per_w
        pltpu.sync_copy(uid_hbm.at[pl.ds(base, b_per_w)], uidx_v)
        pltpu.sync_copy(mid_hbm.at[pl.ds(base, b_per_w)], midx_v)
        cu = pltpu.async_copy(ut_hbm.at[uidx_v], urows_v, usem)
        cm = pltpu.async_copy(mt_hbm.at[midx_v], mrows_v, msem)
        cu.wait()
        cm.wait()
        pltpu.sync_copy(urows_v, uout_hbm.at[pl.ds(base, b_per_w)])
        pltpu.sync_copy(mrows_v, mout_hbm.at[pl.ds(base, b_per_w)])

    return gather


def _mlp(u_emb, m_emb, W1u, W1m, b1r, W2, b2r, w3row, b3s, B, D):
    BLK = 2048
    H1 = W1u.shape[1]
    H2 = W2.shape[1]

    def body(u_ref, m_ref, w1u_ref, w1m_ref, b1_ref, w2_ref, b2_ref,
             w3_ref, b3_ref, o_ref):
        h = (jnp.dot(u_ref[...], w1u_ref[...], preferred_element_type=jnp.float32)
             + jnp.dot(m_ref[...], w1m_ref[...], preferred_element_type=jnp.float32)
             + b1_ref[...])
        h = jnp.maximum(h, 0.0)
        h = jnp.maximum(
            jnp.dot(h, w2_ref[...], preferred_element_type=jnp.float32)
            + b2_ref[...], 0.0)
        o = jnp.sum(h * w3_ref[...], axis=1) + b3_ref[0]
        o_ref[...] = o.reshape(BLK // 128, 128)

    out = pl.pallas_call(
        body,
        grid=(B // BLK,),
        in_specs=[
            pl.BlockSpec((BLK, D), lambda i: (i, 0)),
            pl.BlockSpec((BLK, D), lambda i: (i, 0)),
            pl.BlockSpec((D, H1), lambda i: (0, 0)),
            pl.BlockSpec((D, H1), lambda i: (0, 0)),
            pl.BlockSpec((1, H1), lambda i: (0, 0)),
            pl.BlockSpec((H1, H2), lambda i: (0, 0)),
            pl.BlockSpec((1, H2), lambda i: (0, 0)),
            pl.BlockSpec((1, H2), lambda i: (0, 0)),
            pl.BlockSpec(memory_space=pltpu.SMEM),
        ],
        out_specs=pl.BlockSpec((BLK // 128, 128), lambda i: (i, 0)),
        out_shape=jax.ShapeDtypeStruct((B // 128, 128), jnp.float32),
    )(u_emb, m_emb, W1u, W1m, b1r, W2, b2r, w3row, b3s)
    return out.reshape(B)


def kernel(user_ids, movie_ids, user_table, movie_table, W1, b1, W2, b2, W3, b3):
    B = user_ids.shape[0]
    D = user_table.shape[1]
    info = plsc.get_sparse_core_info()
    gather = _make_gather(B, D, info.num_cores, info.num_subcores)
    u_emb, m_emb = gather(user_ids, movie_ids, user_table, movie_table)
    return _mlp(u_emb, m_emb, W1[:D], W1[D:], b1.reshape(1, -1), W2,
                b2.reshape(1, -1), W3.reshape(1, -1), b3, B, D)
